# Initial kernel scaffold; baseline (speedup 1.0000x reference)
import jax
import jax.numpy as jnp
from jax.experimental import pallas as pl


def kernel(z, edge_index):
    raise NotImplementedError("write your pallas kernel here")


# speedup vs baseline: 4.2180x; 4.2180x over previous
import functools

import jax
import jax.numpy as jnp
from jax import lax
from jax.experimental import pallas as pl
from jax.experimental.pallas import tpu as pltpu
from jax.experimental.pallas import tpu_sc as plsc

_N_NODES = 10000
_D = 128
_E = 320000


def _normalize_body(z_ref, o_ref):
    x = z_ref[:]
    s = jnp.sum(x * x, axis=1, keepdims=True)
    o_ref[:] = x * lax.rsqrt(s)


def _normalize(z):
    return pl.pallas_call(
        _normalize_body,
        out_shape=jax.ShapeDtypeStruct(z.shape, z.dtype),
    )(z)


def _make_edge_kernel():
    info = plsc.get_sparse_core_info()
    nc, ns, lanes = info.num_cores, info.num_subcores, info.num_lanes
    nw = nc * ns
    epw = _E // nw
    B = 16
    nch = epw // B
    groups = _D // lanes

    mesh = plsc.VectorSubcoreMesh(core_axis_name="c", subcore_axis_name="s")

    @functools.partial(
        pl.kernel,
        mesh=mesh,
        compiler_params=pltpu.CompilerParams(needs_layout_passes=False),
        out_type=jax.ShapeDtypeStruct((_E,), jnp.float32),
        scratch_types=[
            pltpu.VMEM((epw,), jnp.int32),
            pltpu.VMEM((epw,), jnp.int32),
            pltpu.VMEM((B, _D), jnp.float32),
            pltpu.VMEM((B, _D), jnp.float32),
            pltpu.VMEM((B, _D), jnp.float32),
            pltpu.VMEM((B, _D), jnp.float32),
            pltpu.VMEM((B, 16), jnp.float32),
            pltpu.VMEM((epw,), jnp.float32),
            pltpu.SemaphoreType.DMA,
            pltpu.SemaphoreType.DMA,
        ],
    )
    def edge_kernel(zn, srci, dsti, out, si, di, rs0, rd0, rs1, rd1, accb,
                    ov, sem0, sem1):
        wid = lax.axis_index("s") * nc + lax.axis_index("c")
        base = pl.multiple_of(wid * epw, 8)

        pltpu.sync_copy(srci.at[pl.ds(base, epw)], si)
        pltpu.sync_copy(dsti.at[pl.ds(base, epw)], di)

        def fire(c, rs, rd, sem):
            start = pl.multiple_of(c * B, 8)
            pltpu.async_copy(zn.at[si.at[pl.ds(start, B)]], rs, sem)
            pltpu.async_copy(zn.at[di.at[pl.ds(start, B)]], rd, sem)

        def drain(c, rs, rd, sem):
            start = pl.multiple_of(c * B, 8)
            pltpu.make_async_copy(zn.at[si.at[pl.ds(start, B)]], rs, sem).wait()
            pltpu.make_async_copy(zn.at[di.at[pl.ds(start, B)]], rd, sem).wait()

        lane_ids = lax.iota(jnp.int32, lanes)

        def compute(c, rs, rd):
            for e in range(B):
                acc = rs[e, 0:lanes] * rd[e, 0:lanes]
                for j in range(1, groups):
                    acc = acc + rs[e, j * lanes:(j + 1) * lanes] * \
                        rd[e, j * lanes:(j + 1) * lanes]
                accb[e, :] = acc
            outv = jnp.zeros((lanes,), jnp.float32)
            for col in range(lanes):
                col_ids = jnp.full((lanes,), col, jnp.int32)
                outv = outv + plsc.load_gather(accb, [lane_ids, col_ids])
            start = pl.multiple_of(c * B, 8)
            ov[pl.ds(start, B)] = outv * 0.5 + 0.5

        fire(0, rs0, rd0, sem0)
        fire(1, rs1, rd1, sem1)

        def loop_body(i, carry):
            c = i * 2
            drain(c, rs0, rd0, sem0)
            compute(c, rs0, rd0)
            fire(c + 2, rs0, rd0, sem0)

            drain(c + 1, rs1, rd1, sem1)
            compute(c + 1, rs1, rd1)

            @pl.when(c + 3 < nch)
            def _():
                fire(c + 3, rs1, rd1, sem1)

            return carry

        lax.fori_loop(0, (nch - 1) // 2, loop_body, 0)

        drain(nch - 1, rs0, rd0, sem0)
        compute(nch - 1, rs0, rd0)

        pltpu.sync_copy(ov, out.at[pl.ds(base, epw)])

    return edge_kernel


_edge_kernel = _make_edge_kernel()


def kernel(z, edge_index):
    ei = edge_index.astype(jnp.int32)
    zn = _normalize(z)
    return _edge_kernel(zn, ei[0], ei[1])


# speedup vs baseline: 5.5028x; 1.3046x over previous
import functools

import jax
import jax.numpy as jnp
from jax import lax
from jax.experimental import pallas as pl
from jax.experimental.pallas import tpu as pltpu
from jax.experimental.pallas import tpu_sc as plsc

_N_NODES = 10000
_D = 128
_E = 320000


def _normalize_body(z_ref, o_ref):
    x = z_ref[:]
    s = jnp.sum(x * x, axis=1, keepdims=True)
    o_ref[:] = (x * lax.rsqrt(s)).astype(jnp.bfloat16)


def _normalize(z):
    return pl.pallas_call(
        _normalize_body,
        out_shape=jax.ShapeDtypeStruct(z.shape, jnp.bfloat16),
    )(z)


def _make_edge_kernel():
    info = plsc.get_sparse_core_info()
    nc, ns, lanes = info.num_cores, info.num_subcores, info.num_lanes
    nw = nc * ns
    epw = _E // nw
    B = 16
    nch = epw // B
    groups = _D // lanes

    mesh = plsc.VectorSubcoreMesh(core_axis_name="c", subcore_axis_name="s")

    @functools.partial(
        pl.kernel,
        mesh=mesh,
        compiler_params=pltpu.CompilerParams(
            needs_layout_passes=False, use_tc_tiling_on_sc=False),
        out_type=jax.ShapeDtypeStruct((_E,), jnp.float32),
        scratch_types=[
            pltpu.VMEM((epw,), jnp.int32),
            pltpu.VMEM((epw,), jnp.int32),
            pltpu.VMEM((B, _D // 2), jnp.int32),
            pltpu.VMEM((B, _D // 2), jnp.int32),
            pltpu.VMEM((B, _D // 2), jnp.int32),
            pltpu.VMEM((B, _D // 2), jnp.int32),
            pltpu.VMEM((epw,), jnp.float32),
            pltpu.SemaphoreType.DMA,
            pltpu.SemaphoreType.DMA,
        ],
    )
    def edge_kernel(zn, srci, dsti, out, si, di, rs0, rd0, rs1, rd1,
                    ov, sem0, sem1):
        wid = lax.axis_index("s") * nc + lax.axis_index("c")
        base = pl.multiple_of(wid * epw, 8)

        pltpu.sync_copy(srci.at[pl.ds(base, epw)], si)
        pltpu.sync_copy(dsti.at[pl.ds(base, epw)], di)

        def fire(c, rs, rd, sem):
            start = pl.multiple_of(c * B, 8)
            pltpu.async_copy(zn.at[si.at[pl.ds(start, B)]], rs, sem)
            pltpu.async_copy(zn.at[di.at[pl.ds(start, B)]], rd, sem)

        def drain(c, rs, rd, sem):
            start = pl.multiple_of(c * B, 8)
            pltpu.make_async_copy(zn.at[si.at[pl.ds(start, B)]], rs, sem).wait()
            pltpu.make_async_copy(zn.at[di.at[pl.ds(start, B)]], rd, sem).wait()

        lane_ids = lax.iota(jnp.int32, lanes)

        bgroups = _D // (2 * lanes)

        def compute(c, rs, rd):
            outv = jnp.zeros((lanes,), jnp.float32)
            for e in range(B):
                acc = None
                for j in range(bgroups):
                    s = plsc.bitcast(rs[e, j * lanes:(j + 1) * lanes],
                                     jnp.bfloat16)
                    d = plsc.bitcast(rd[e, j * lanes:(j + 1) * lanes],
                                     jnp.bfloat16)
                    p = s * d
                    acc = p if acc is None else acc + p
                a, b = plsc.unpack(acc, format=plsc.PackFormat.INTERLEAVED)
                tot = jnp.sum(a + b)
                outv = jnp.where(lane_ids == e, tot, outv)
            start = pl.multiple_of(c * B, 8)
            ov[pl.ds(start, B)] = outv * 0.5 + 0.5

        fire(0, rs0, rd0, sem0)
        fire(1, rs1, rd1, sem1)

        def loop_body(i, carry):
            c = i * 2
            drain(c, rs0, rd0, sem0)
            compute(c, rs0, rd0)
            fire(c + 2, rs0, rd0, sem0)

            drain(c + 1, rs1, rd1, sem1)
            compute(c + 1, rs1, rd1)

            @pl.when(c + 3 < nch)
            def _():
                fire(c + 3, rs1, rd1, sem1)

            return carry

        lax.fori_loop(0, (nch - 1) // 2, loop_body, 0)

        drain(nch - 1, rs0, rd0, sem0)
        compute(nch - 1, rs0, rd0)

        pltpu.sync_copy(ov, out.at[pl.ds(base, epw)])

    return edge_kernel


_edge_kernel = _make_edge_kernel()


def kernel(z, edge_index):
    ei = edge_index.astype(jnp.int32)
    zn = _normalize(z)
    zn_i32 = lax.bitcast_convert_type(
        zn.reshape(_N_NODES, _D // 2, 2), jnp.int32)
    return _edge_kernel(zn_i32, ei[0], ei[1])


# speedup vs baseline: 7.4734x; 1.3581x over previous
import functools

import jax
import jax.numpy as jnp
from jax import lax
from jax.experimental import pallas as pl
from jax.experimental.pallas import tpu as pltpu
from jax.experimental.pallas import tpu_sc as plsc

_N_NODES = 10000
_D = 128
_E = 320000


def _normalize_body(z_ref, o_ref):
    x = z_ref[:]
    s = jnp.sum(x * x, axis=1, keepdims=True)
    o_ref[:] = (x * lax.rsqrt(s)).astype(jnp.bfloat16)


def _normalize(z):
    return pl.pallas_call(
        _normalize_body,
        out_shape=jax.ShapeDtypeStruct(z.shape, jnp.bfloat16),
    )(z)


def _make_edge_kernel():
    info = plsc.get_sparse_core_info()
    nc, ns, lanes = info.num_cores, info.num_subcores, info.num_lanes
    nw = nc * ns
    epw = _E // nw
    B = 80
    nch = epw // B
    groups = _D // lanes

    mesh = plsc.VectorSubcoreMesh(core_axis_name="c", subcore_axis_name="s")

    @functools.partial(
        pl.kernel,
        mesh=mesh,
        compiler_params=pltpu.CompilerParams(
            needs_layout_passes=False, use_tc_tiling_on_sc=False),
        out_type=jax.ShapeDtypeStruct((_E,), jnp.float32),
        scratch_types=[
            pltpu.VMEM((epw,), jnp.int32),
            pltpu.VMEM((epw,), jnp.int32),
            pltpu.VMEM((B, _D // 2), jnp.int32),
            pltpu.VMEM((B, _D // 2), jnp.int32),
            pltpu.VMEM((B, _D // 2), jnp.int32),
            pltpu.VMEM((B, _D // 2), jnp.int32),
            pltpu.VMEM((epw,), jnp.float32),
            pltpu.SemaphoreType.DMA,
            pltpu.SemaphoreType.DMA,
        ],
    )
    def edge_kernel(zn, srci, dsti, out, si, di, rs0, rd0, rs1, rd1,
                    ov, sem0, sem1):
        wid = lax.axis_index("s") * nc + lax.axis_index("c")
        base = pl.multiple_of(wid * epw, 8)

        pltpu.sync_copy(srci.at[pl.ds(base, epw)], si)
        pltpu.sync_copy(dsti.at[pl.ds(base, epw)], di)

        def fire(c, rs, rd, sem):
            start = pl.multiple_of(c * B, 8)
            pltpu.async_copy(zn.at[si.at[pl.ds(start, B)]], rs, sem)
            pltpu.async_copy(zn.at[di.at[pl.ds(start, B)]], rd, sem)

        def drain(c, rs, rd, sem):
            start = pl.multiple_of(c * B, 8)
            pltpu.make_async_copy(zn.at[si.at[pl.ds(start, B)]], rs, sem).wait()
            pltpu.make_async_copy(zn.at[di.at[pl.ds(start, B)]], rd, sem).wait()

        lane_ids = lax.iota(jnp.int32, lanes)

        bgroups = _D // (2 * lanes)

        def compute(c, rs, rd):
            start = pl.multiple_of(c * B, 8)
            for g in range(B // lanes):
                outv = jnp.zeros((lanes,), jnp.float32)
                for l in range(lanes):
                    e = g * lanes + l
                    acc = None
                    for j in range(bgroups):
                        s = plsc.bitcast(rs[e, j * lanes:(j + 1) * lanes],
                                         jnp.bfloat16)
                        d = plsc.bitcast(rd[e, j * lanes:(j + 1) * lanes],
                                         jnp.bfloat16)
                        p = s * d
                        acc = p if acc is None else acc + p
                    a, b = plsc.unpack(acc, format=plsc.PackFormat.INTERLEAVED)
                    tot = jnp.sum(a + b)
                    outv = jnp.where(lane_ids == l, tot, outv)
                ov[pl.ds(start + g * lanes, lanes)] = outv * 0.5 + 0.5

        fire(0, rs0, rd0, sem0)
        fire(1, rs1, rd1, sem1)

        def loop_body(i, carry):
            c = i * 2
            drain(c, rs0, rd0, sem0)
            compute(c, rs0, rd0)

            @pl.when(c + 2 < nch)
            def _():
                fire(c + 2, rs0, rd0, sem0)

            drain(c + 1, rs1, rd1, sem1)
            compute(c + 1, rs1, rd1)

            @pl.when(c + 3 < nch)
            def _():
                fire(c + 3, rs1, rd1, sem1)

            return carry

        lax.fori_loop(0, nch // 2, loop_body, 0)

        if nch % 2:
            drain(nch - 1, rs0, rd0, sem0)
            compute(nch - 1, rs0, rd0)

        pltpu.sync_copy(ov, out.at[pl.ds(base, epw)])

    return edge_kernel


_edge_kernel = _make_edge_kernel()


def kernel(z, edge_index):
    ei = edge_index.astype(jnp.int32)
    zn = _normalize(z)
    zn_i32 = lax.bitcast_convert_type(
        zn.reshape(_N_NODES, _D // 2, 2), jnp.int32)
    return _edge_kernel(zn_i32, ei[0], ei[1])
